# baseline (speedup 1.0000x reference)
import functools

import numpy as np
import jax
import jax.numpy as jnp
from jax import lax
from jax.experimental import pallas as pl
from jax.experimental.pallas import tpu as pltpu
from jax.experimental.pallas import tpu_sc as plsc

N_COARSE = 12500
N_FINE = 25000
D = 256
K = 9
CG = 8
EPS = 1e-5

RB = 512
NCP = 12800
NRB = NCP // RB

NW = 32
L = 16
SC_C = 112
NCHUNK = -(-N_FINE // SC_C)
TAIL = N_FINE - (NCHUNK - 1) * SC_C
CPW = -(-NCHUNK // NW)

NIDX = NCHUNK * SC_C * K
IROWS = -(-NIDX // 128)
IPAD = IROWS * 128


def _bf16_bits(x):
    u = lax.bitcast_convert_type(x, jnp.uint32)
    return (u + R_BIAS + ((u >> U16) & U1)) >> U16


def _tc_body(x_ref, g_ref, b_ref, we_ref, wo_ref, i2_ref, out_ref, o2_ref,
             sb_ref):
    i = pl.program_id(0)

    @pl.when(i == 0)
    def _stats():
        x = x_ref[...]
        s = jnp.sum(x, axis=0, keepdims=True)
        q = jnp.sum(x * x, axis=0, keepdims=True)
        row = lax.broadcasted_iota(jnp.int32, (D, D), 0) // CG
        col = lax.broadcasted_iota(jnp.int32, (D, D), 1) // CG
        m = (row == col).astype(jnp.float32)
        cnt = float(CG * N_COARSE)
        gs = jnp.dot(s, m, preferred_element_type=jnp.float32) / cnt
        gq = jnp.dot(q, m, preferred_element_type=jnp.float32) / cnt
        var = gq - gs * gs
        scale = g_ref[...] * lax.rsqrt(var + EPS)
        bias = b_ref[...] - gs * scale
        sb_ref[0:1, :] = scale
        sb_ref[1:2, :] = bias
        pos = (lax.broadcasted_iota(jnp.int32, (IROWS, 128), 0) * 128
               + lax.broadcasted_iota(jnp.int32, (IROWS, 128), 1))
        o2_ref[...] = i2_ref[...] + ((pos // SC_C) % K) * NCP

    scale = sb_ref[0:1, :]
    bias = sb_ref[1:2, :]
    x = x_ref[pl.ds(i * RB, RB), :]
    y = jnp.maximum(x * scale + bias, 0.0).astype(jnp.bfloat16)
    for k in range(K):
        re = jnp.dot(y, we_ref[pl.ds(k * D, D), :],
                     preferred_element_type=jnp.float32)
        ro = jnp.dot(y, wo_ref[pl.ds(k * D, D), :],
                     preferred_element_type=jnp.float32)
        packed = _bf16_bits(re) | (_bf16_bits(ro) << 16)
        out_ref[k] = lax.bitcast_convert_type(packed, jnp.int32)


DH = D // 2
M_HI = np.uint32(0xFFFF0000)
R_BIAS = np.uint32(0x7FFF)
U1 = np.uint32(1)
U16 = np.uint32(16)


def _sc_body(tab_hbm, idx_hbm, out_hbm, ib, gb0, gb1, gb2, acc, sem0, sem1,
             sem2):
    wid = lax.axis_index("s") * 2 + lax.axis_index("c")
    gbs = (gb0, gb1, gb2)
    sems = (sem0, sem1, sem2)

    def _unpack(v):
        u = lax.bitcast_convert_type(v, jnp.uint32)
        lo = lax.bitcast_convert_type(u << U16, jnp.float32)
        hi = lax.bitcast_convert_type(u & M_HI, jnp.float32)
        return lo, hi

    def _accum(buf, first):
        def _row(r, c):
            for h in range(DH // L):
                sl = pl.ds(h * L, L)
                lo, hi = _unpack(buf[r, sl])
                if first:
                    acc[r, sl] = lo
                    acc[r, pl.ds(DH + h * L, L)] = hi
                else:
                    plsc.addupdate(acc.at[r, sl], lo)
                    plsc.addupdate(acc.at[r, pl.ds(DH + h * L, L)], hi)
            return c

        lax.fori_loop(0, SC_C, _row, 0)

    def _repack(dst):
        def _row(r, c):
            for h in range(DH // L):
                sl = pl.ds(h * L, L)
                lo = lax.bitcast_convert_type(acc[r, sl], jnp.uint32)
                hi = lax.bitcast_convert_type(acc[r, pl.ds(DH + h * L, L)],
                                              jnp.uint32)
                lo = (lo + R_BIAS + ((lo >> U16) & U1)) >> U16
                hi = (hi + R_BIAS + ((hi >> U16) & U1)) & M_HI
                dst[r, sl] = lax.bitcast_convert_type(lo | hi, jnp.int32)
            return c

        lax.fori_loop(0, SC_C, _row, 0)

    def chunk_body(ci, carry):
        chunk = wid + ci * NW

        @pl.when(chunk < NCHUNK)
        def _():
            for k in range(K):
                pltpu.async_copy(
                    idx_hbm.at[pl.ds(chunk * K * SC_C + k * SC_C, SC_C)],
                    ib.at[k], sem0)
            for k in range(K):
                pltpu.make_async_copy(
                    idx_hbm.at[pl.ds(chunk * K * SC_C + k * SC_C, SC_C)],
                    ib.at[k], sem0).wait()
            for k in range(3):
                pltpu.async_copy(tab_hbm.at[ib.at[k]], gbs[k], sems[k])
            for k in range(K):
                b = k % 3
                pltpu.make_async_copy(tab_hbm.at[ib.at[k]], gbs[b],
                                      sems[b]).wait()
                _accum(gbs[b], first=(k == 0))
                if k + 3 < K:
                    pltpu.async_copy(tab_hbm.at[ib.at[k + 3]], gbs[b],
                                     sems[b])
            _repack(gb0)
            base = chunk * SC_C

            @pl.when(chunk < NCHUNK - 1)
            def _full():
                pltpu.sync_copy(gb0, out_hbm.at[pl.ds(base, SC_C)])

            @pl.when(chunk == NCHUNK - 1)
            def _tail():
                pltpu.sync_copy(gb0.at[pl.ds(0, TAIL)],
                                out_hbm.at[pl.ds(base, TAIL)])

        return carry

    lax.fori_loop(0, CPW, chunk_body, 0)


def kernel(lv_coarse, neighbor_idx, gn_gamma, gn_beta, weight):
    lv_pad = jnp.pad(lv_coarse, ((0, NCP - N_COARSE), (0, 0)))
    idx_pad = jnp.pad(neighbor_idx, ((0, NCHUNK * SC_C - N_FINE), (0, 0)))
    idx_cm = idx_pad.reshape(NCHUNK, SC_C, K).transpose(0, 2, 1).reshape(-1)
    idx_2d = jnp.pad(idx_cm, (0, IPAD - NIDX)).reshape(IROWS, 128)

    table, flat_idx = pl.pallas_call(
        _tc_body,
        grid=(NRB,),
        in_specs=[
            pl.BlockSpec((NCP, D), lambda i: (0, 0)),
            pl.BlockSpec((1, D), lambda i: (0, 0)),
            pl.BlockSpec((1, D), lambda i: (0, 0)),
            pl.BlockSpec((K * D, D // 2), lambda i: (0, 0)),
            pl.BlockSpec((K * D, D // 2), lambda i: (0, 0)),
            pl.BlockSpec((IROWS, 128), lambda i: (0, 0)),
        ],
        out_specs=[
            pl.BlockSpec((K, RB, D // 2), lambda i: (0, i, 0)),
            pl.BlockSpec((IROWS, 128), lambda i: (0, 0)),
        ],
        out_shape=[
            jax.ShapeDtypeStruct((K, NCP, D // 2), jnp.int32),
            jax.ShapeDtypeStruct((IROWS, 128), jnp.int32),
        ],
        scratch_shapes=[
            pltpu.VMEM((2, D), jnp.float32),
        ],
    )(lv_pad, gn_gamma.reshape(1, D), gn_beta.reshape(1, D),
      weight.astype(jnp.bfloat16)[:, 0::2],
      weight.astype(jnp.bfloat16)[:, 1::2], idx_2d)

    tab_flat = table.reshape(K * NCP, DH)
    flat_idx = flat_idx.reshape(IPAD)

    mesh = plsc.VectorSubcoreMesh(core_axis_name="c", subcore_axis_name="s")
    out = pl.kernel(
        _sc_body,
        out_type=jax.ShapeDtypeStruct((N_FINE, DH), jnp.int32),
        mesh=mesh,
        scratch_types=[
            pltpu.VMEM((K, SC_C), jnp.int32),
            pltpu.VMEM((SC_C, DH), jnp.int32),
            pltpu.VMEM((SC_C, DH), jnp.int32),
            pltpu.VMEM((SC_C, DH), jnp.int32),
            pltpu.VMEM((SC_C, D), jnp.float32),
            pltpu.SemaphoreType.DMA,
            pltpu.SemaphoreType.DMA,
            pltpu.SemaphoreType.DMA,
        ],
    )(tab_flat, flat_idx)
    out_bf = lax.bitcast_convert_type(out, jnp.bfloat16)
    return out_bf.reshape(N_FINE, D).astype(jnp.float32)


# speedup vs baseline: 1.4447x; 1.4447x over previous
import functools

import numpy as np
import jax
import jax.numpy as jnp
from jax import lax
from jax.experimental import pallas as pl
from jax.experimental.pallas import tpu as pltpu
from jax.experimental.pallas import tpu_sc as plsc

N_COARSE = 12500
N_FINE = 25000
D = 256
K = 9
CG = 8
EPS = 1e-5

RB = 512
NCP = 12800
NRB = NCP // RB

NW = 32
L = 16
SC_C = 112
NCHUNK = -(-N_FINE // SC_C)
TAIL = N_FINE - (NCHUNK - 1) * SC_C
CPW = -(-NCHUNK // NW)

NIDX = NCHUNK * SC_C * K
IROWS = -(-NIDX // 128)
IPAD = IROWS * 128


def _bf16_bits(x):
    u = lax.bitcast_convert_type(x, jnp.uint32)
    return (u + R_BIAS + ((u >> U16) & U1)) >> U16


def _tc_body(x_ref, g_ref, b_ref, we_ref, wo_ref, i2_ref, out_ref, o2_ref,
             sb_ref):
    i = pl.program_id(0)

    @pl.when(i == 0)
    def _stats():
        x = x_ref[...]
        s = jnp.sum(x, axis=0, keepdims=True)
        q = jnp.sum(x * x, axis=0, keepdims=True)
        row = lax.broadcasted_iota(jnp.int32, (D, D), 0) // CG
        col = lax.broadcasted_iota(jnp.int32, (D, D), 1) // CG
        m = (row == col).astype(jnp.float32)
        cnt = float(CG * N_COARSE)
        gs = jnp.dot(s, m, preferred_element_type=jnp.float32) / cnt
        gq = jnp.dot(q, m, preferred_element_type=jnp.float32) / cnt
        var = gq - gs * gs
        scale = g_ref[...] * lax.rsqrt(var + EPS)
        bias = b_ref[...] - gs * scale
        sb_ref[0:1, :] = scale
        sb_ref[1:2, :] = bias
        pos = (lax.broadcasted_iota(jnp.int32, (IROWS, 128), 0) * 128
               + lax.broadcasted_iota(jnp.int32, (IROWS, 128), 1))
        o2_ref[...] = i2_ref[...] + ((pos // SC_C) % K) * NCP

    scale = sb_ref[0:1, :]
    bias = sb_ref[1:2, :]
    x = x_ref[pl.ds(i * RB, RB), :]
    y = jnp.maximum(x * scale + bias, 0.0).astype(jnp.bfloat16)
    for k in range(K):
        re = jnp.dot(y, we_ref[pl.ds(k * D, D), :],
                     preferred_element_type=jnp.float32)
        ro = jnp.dot(y, wo_ref[pl.ds(k * D, D), :],
                     preferred_element_type=jnp.float32)
        packed = _bf16_bits(re) | (_bf16_bits(ro) << 16)
        out_ref[k] = lax.bitcast_convert_type(packed, jnp.int32)


DH = D // 2
M_HI = np.uint32(0xFFFF0000)
R_BIAS = np.uint32(0x7FFF)
U1 = np.uint32(1)
U16 = np.uint32(16)


def _sc_body(tab_hbm, idx_hbm, out_hbm, ib, gb0, gb1, gb2, acc, sem0, sem1,
             sem2):
    wid = lax.axis_index("s") * 2 + lax.axis_index("c")
    gbs = (gb0, gb1, gb2)
    sems = (sem0, sem1, sem2)

    def _unpack(v):
        u = lax.bitcast_convert_type(v, jnp.uint32)
        lo = lax.bitcast_convert_type(u << U16, jnp.float32)
        hi = lax.bitcast_convert_type(u & M_HI, jnp.float32)
        return lo, hi

    def _accum(buf, first):
        def _row(r):
            for h in range(DH // L):
                sl = pl.ds(h * L, L)
                lo, hi = _unpack(buf[r, sl])
                if first:
                    acc[r, sl] = lo
                    acc[r, pl.ds(DH + h * L, L)] = hi
                else:
                    plsc.addupdate(acc.at[r, sl], lo)
                    plsc.addupdate(acc.at[r, pl.ds(DH + h * L, L)], hi)

        plsc.parallel_loop(0, SC_C, unroll=4)(_row)

    def _repack(dst):
        def _row(r):
            for h in range(DH // L):
                sl = pl.ds(h * L, L)
                lo = lax.bitcast_convert_type(acc[r, sl], jnp.uint32)
                hi = lax.bitcast_convert_type(acc[r, pl.ds(DH + h * L, L)],
                                              jnp.uint32)
                lo = (lo + R_BIAS + ((lo >> U16) & U1)) >> U16
                hi = (hi + R_BIAS + ((hi >> U16) & U1)) & M_HI
                dst[r, sl] = lax.bitcast_convert_type(lo | hi, jnp.int32)

        plsc.parallel_loop(0, SC_C, unroll=2)(_row)

    def chunk_body(ci, carry):
        chunk = wid + ci * NW

        @pl.when(chunk < NCHUNK)
        def _():
            for k in range(K):
                pltpu.async_copy(
                    idx_hbm.at[pl.ds(chunk * K * SC_C + k * SC_C, SC_C)],
                    ib.at[k], sem0)
            for k in range(K):
                pltpu.make_async_copy(
                    idx_hbm.at[pl.ds(chunk * K * SC_C + k * SC_C, SC_C)],
                    ib.at[k], sem0).wait()
            for k in range(3):
                pltpu.async_copy(tab_hbm.at[ib.at[k]], gbs[k], sems[k])
            for k in range(K):
                b = k % 3
                pltpu.make_async_copy(tab_hbm.at[ib.at[k]], gbs[b],
                                      sems[b]).wait()
                _accum(gbs[b], first=(k == 0))
                if k + 3 < K:
                    pltpu.async_copy(tab_hbm.at[ib.at[k + 3]], gbs[b],
                                     sems[b])
            _repack(gb0)
            base = chunk * SC_C

            @pl.when(chunk < NCHUNK - 1)
            def _full():
                pltpu.sync_copy(gb0, out_hbm.at[pl.ds(base, SC_C)])

            @pl.when(chunk == NCHUNK - 1)
            def _tail():
                pltpu.sync_copy(gb0.at[pl.ds(0, TAIL)],
                                out_hbm.at[pl.ds(base, TAIL)])

        return carry

    lax.fori_loop(0, CPW, chunk_body, 0)


def kernel(lv_coarse, neighbor_idx, gn_gamma, gn_beta, weight):
    lv_pad = jnp.pad(lv_coarse, ((0, NCP - N_COARSE), (0, 0)))
    idx_pad = jnp.pad(neighbor_idx, ((0, NCHUNK * SC_C - N_FINE), (0, 0)))
    idx_cm = idx_pad.reshape(NCHUNK, SC_C, K).transpose(0, 2, 1).reshape(-1)
    idx_2d = jnp.pad(idx_cm, (0, IPAD - NIDX)).reshape(IROWS, 128)

    table, flat_idx = pl.pallas_call(
        _tc_body,
        grid=(NRB,),
        in_specs=[
            pl.BlockSpec((NCP, D), lambda i: (0, 0)),
            pl.BlockSpec((1, D), lambda i: (0, 0)),
            pl.BlockSpec((1, D), lambda i: (0, 0)),
            pl.BlockSpec((K * D, D // 2), lambda i: (0, 0)),
            pl.BlockSpec((K * D, D // 2), lambda i: (0, 0)),
            pl.BlockSpec((IROWS, 128), lambda i: (0, 0)),
        ],
        out_specs=[
            pl.BlockSpec((K, RB, D // 2), lambda i: (0, i, 0)),
            pl.BlockSpec((IROWS, 128), lambda i: (0, 0)),
        ],
        out_shape=[
            jax.ShapeDtypeStruct((K, NCP, D // 2), jnp.int32),
            jax.ShapeDtypeStruct((IROWS, 128), jnp.int32),
        ],
        scratch_shapes=[
            pltpu.VMEM((2, D), jnp.float32),
        ],
    )(lv_pad, gn_gamma.reshape(1, D), gn_beta.reshape(1, D),
      weight.astype(jnp.bfloat16)[:, 0::2],
      weight.astype(jnp.bfloat16)[:, 1::2], idx_2d)

    tab_flat = table.reshape(K * NCP, DH)
    flat_idx = flat_idx.reshape(IPAD)

    mesh = plsc.VectorSubcoreMesh(core_axis_name="c", subcore_axis_name="s")
    out = pl.kernel(
        _sc_body,
        out_type=jax.ShapeDtypeStruct((N_FINE, DH), jnp.int32),
        mesh=mesh,
        scratch_types=[
            pltpu.VMEM((K, SC_C), jnp.int32),
            pltpu.VMEM((SC_C, DH), jnp.int32),
            pltpu.VMEM((SC_C, DH), jnp.int32),
            pltpu.VMEM((SC_C, DH), jnp.int32),
            pltpu.VMEM((SC_C, D), jnp.float32),
            pltpu.SemaphoreType.DMA,
            pltpu.SemaphoreType.DMA,
            pltpu.SemaphoreType.DMA,
        ],
    )(tab_flat, flat_idx)
    out_bf = lax.bitcast_convert_type(out, jnp.bfloat16)
    return out_bf.reshape(N_FINE, D).astype(jnp.float32)


# speedup vs baseline: 2.2320x; 1.5450x over previous
import functools

import jax
import jax.numpy as jnp
from jax import lax
from jax.experimental import pallas as pl
from jax.experimental.pallas import tpu as pltpu
from jax.experimental.pallas import tpu_sc as plsc

N_COARSE = 12500
N_FINE = 25000
D = 256
K = 9
CG = 8
EPS = 1e-5

RB = 512
NCP = 12800
NRB = NCP // RB

NW = 32
L = 16
SC_C = 112
NCHUNK = -(-N_FINE // SC_C)
TAIL = N_FINE - (NCHUNK - 1) * SC_C
CPW = -(-NCHUNK // NW)

NIDX = NCHUNK * SC_C * K
IROWS = -(-NIDX // 128)
IPAD = IROWS * 128


def _tc_body(x_ref, g_ref, b_ref, w_ref, i2_ref, out_ref, o2_ref, sb_ref):
    i = pl.program_id(0)

    @pl.when(i == 0)
    def _stats():
        x = x_ref[...]
        s = jnp.sum(x, axis=0, keepdims=True)
        q = jnp.sum(x * x, axis=0, keepdims=True)
        row = lax.broadcasted_iota(jnp.int32, (D, D), 0) // CG
        col = lax.broadcasted_iota(jnp.int32, (D, D), 1) // CG
        m = (row == col).astype(jnp.float32)
        cnt = float(CG * N_COARSE)
        gs = jnp.dot(s, m, preferred_element_type=jnp.float32) / cnt
        gq = jnp.dot(q, m, preferred_element_type=jnp.float32) / cnt
        var = gq - gs * gs
        scale = g_ref[...] * lax.rsqrt(var + EPS)
        bias = b_ref[...] - gs * scale
        sb_ref[0:1, :] = scale
        sb_ref[1:2, :] = bias
        pos = (lax.broadcasted_iota(jnp.int32, (IROWS, 128), 0) * 128
               + lax.broadcasted_iota(jnp.int32, (IROWS, 128), 1))
        o2_ref[...] = i2_ref[...] + ((pos // SC_C) % K) * NCP

    scale = sb_ref[0:1, :]
    bias = sb_ref[1:2, :]
    x = x_ref[pl.ds(i * RB, RB), :]
    y = jnp.maximum(x * scale + bias, 0.0)
    for k in range(K):
        out_ref[k] = jnp.dot(y, w_ref[pl.ds(k * D, D), :],
                             preferred_element_type=jnp.float32)


def _sc_body(tab_hbm, idx_hbm, out_hbm, ib, gb0, gb1, gb2, acc, sem0, sem1,
             sem2, sem3):
    wid = lax.axis_index("s") * 2 + lax.axis_index("c")
    gbs = (gb0, gb1, gb2)
    sems = (sem1, sem2, sem3)

    def _accum(buf):
        def _row(r):
            for cc in range(D // L):
                plsc.addupdate(acc.at[r, pl.ds(cc * L, L)],
                               buf[r, pl.ds(cc * L, L)])

        plsc.parallel_loop(0, SC_C, unroll=4)(_row)

    def chunk_body(ci, carry):
        chunk = wid + ci * NW

        @pl.when(chunk < NCHUNK)
        def _():
            for k in range(K):
                pltpu.async_copy(
                    idx_hbm.at[pl.ds(chunk * K * SC_C + k * SC_C, SC_C)],
                    ib.at[k], sem0)
            for k in range(K):
                pltpu.make_async_copy(
                    idx_hbm.at[pl.ds(chunk * K * SC_C + k * SC_C, SC_C)],
                    ib.at[k], sem0).wait()
            pltpu.async_copy(tab_hbm.at[ib.at[0]], acc, sem0)
            for k in range(1, 4):
                pltpu.async_copy(tab_hbm.at[ib.at[k]], gbs[k - 1],
                                 sems[k - 1])
            pltpu.make_async_copy(tab_hbm.at[ib.at[0]], acc, sem0).wait()
            for k in range(1, K):
                b = (k - 1) % 3
                pltpu.make_async_copy(tab_hbm.at[ib.at[k]], gbs[b],
                                      sems[b]).wait()
                _accum(gbs[b])
                if k + 3 < K:
                    pltpu.async_copy(tab_hbm.at[ib.at[k + 3]], gbs[b],
                                     sems[b])
            base = chunk * SC_C

            @pl.when(chunk < NCHUNK - 1)
            def _full():
                pltpu.sync_copy(acc, out_hbm.at[pl.ds(base, SC_C)])

            @pl.when(chunk == NCHUNK - 1)
            def _tail():
                pltpu.sync_copy(acc.at[pl.ds(0, TAIL)],
                                out_hbm.at[pl.ds(base, TAIL)])

        return carry

    lax.fori_loop(0, CPW, chunk_body, 0)


def kernel(lv_coarse, neighbor_idx, gn_gamma, gn_beta, weight):
    lv_pad = jnp.pad(lv_coarse, ((0, NCP - N_COARSE), (0, 0)))
    idx_pad = jnp.pad(neighbor_idx, ((0, NCHUNK * SC_C - N_FINE), (0, 0)))
    idx_cm = idx_pad.reshape(NCHUNK, SC_C, K).transpose(0, 2, 1).reshape(-1)
    idx_2d = jnp.pad(idx_cm, (0, IPAD - NIDX)).reshape(IROWS, 128)

    table, flat_idx = pl.pallas_call(
        _tc_body,
        grid=(NRB,),
        in_specs=[
            pl.BlockSpec((NCP, D), lambda i: (0, 0)),
            pl.BlockSpec((1, D), lambda i: (0, 0)),
            pl.BlockSpec((1, D), lambda i: (0, 0)),
            pl.BlockSpec((K * D, D), lambda i: (0, 0)),
            pl.BlockSpec((IROWS, 128), lambda i: (0, 0)),
        ],
        out_specs=[
            pl.BlockSpec((K, RB, D), lambda i: (0, i, 0)),
            pl.BlockSpec((IROWS, 128), lambda i: (0, 0)),
        ],
        out_shape=[
            jax.ShapeDtypeStruct((K, NCP, D), jnp.float32),
            jax.ShapeDtypeStruct((IROWS, 128), jnp.int32),
        ],
        scratch_shapes=[
            pltpu.VMEM((2, D), jnp.float32),
        ],
    )(lv_pad, gn_gamma.reshape(1, D), gn_beta.reshape(1, D), weight, idx_2d)

    tab_flat = table.reshape(K * NCP, D)
    flat_idx = flat_idx.reshape(IPAD)

    mesh = plsc.VectorSubcoreMesh(core_axis_name="c", subcore_axis_name="s")
    out = pl.kernel(
        _sc_body,
        out_type=jax.ShapeDtypeStruct((N_FINE, D), jnp.float32),
        mesh=mesh,
        scratch_types=[
            pltpu.VMEM((K, SC_C), jnp.int32),
            pltpu.VMEM((SC_C, D), jnp.float32),
            pltpu.VMEM((SC_C, D), jnp.float32),
            pltpu.VMEM((SC_C, D), jnp.float32),
            pltpu.VMEM((SC_C, D), jnp.float32),
            pltpu.SemaphoreType.DMA,
            pltpu.SemaphoreType.DMA,
            pltpu.SemaphoreType.DMA,
            pltpu.SemaphoreType.DMA,
        ],
    )(tab_flat, flat_idx)
    return out


# speedup vs baseline: 2.3146x; 1.0370x over previous
import functools

import jax
import jax.numpy as jnp
from jax import lax
from jax.experimental import pallas as pl
from jax.experimental.pallas import tpu as pltpu
from jax.experimental.pallas import tpu_sc as plsc

N_COARSE = 12500
N_FINE = 25000
D = 256
K = 9
CG = 8
EPS = 1e-5

RB = 512
NCP = 12800
NRB = NCP // RB

NW = 32
L = 16
SC_C = 112
NCHUNK = -(-N_FINE // SC_C)
TAIL = N_FINE - (NCHUNK - 1) * SC_C
CPW = -(-NCHUNK // NW)

NIDX = NCHUNK * SC_C * K
IROWS = -(-NIDX // 128)
IPAD = IROWS * 128


def _tc_body(x_ref, g_ref, b_ref, w_ref, i2_ref, out_ref, o2_ref, sb_ref):
    i = pl.program_id(0)

    @pl.when(i == 0)
    def _stats():
        rows = lax.broadcasted_iota(jnp.int32, (NCP, 1), 0)
        x = jnp.where(rows < N_COARSE, x_ref[...], 0.0)
        s = jnp.sum(x, axis=0, keepdims=True)
        q = jnp.sum(x * x, axis=0, keepdims=True)
        row = lax.broadcasted_iota(jnp.int32, (D, D), 0) // CG
        col = lax.broadcasted_iota(jnp.int32, (D, D), 1) // CG
        m = (row == col).astype(jnp.float32)
        cnt = float(CG * N_COARSE)
        gs = jnp.dot(s, m, preferred_element_type=jnp.float32) / cnt
        gq = jnp.dot(q, m, preferred_element_type=jnp.float32) / cnt
        var = gq - gs * gs
        scale = g_ref[...] * lax.rsqrt(var + EPS)
        bias = b_ref[...] - gs * scale
        sb_ref[0:1, :] = scale
        sb_ref[1:2, :] = bias
        pos = (lax.broadcasted_iota(jnp.int32, (IROWS, 128), 0) * 128
               + lax.broadcasted_iota(jnp.int32, (IROWS, 128), 1))
        o2_ref[...] = i2_ref[...] + ((pos // SC_C) % K) * NCP

    scale = sb_ref[0:1, :]
    bias = sb_ref[1:2, :]
    x = x_ref[pl.ds(i * RB, RB), :]
    y = jnp.maximum(x * scale + bias, 0.0).astype(jnp.bfloat16)
    for k in range(K):
        out_ref[k] = jnp.dot(y, w_ref[pl.ds(k * D, D), :],
                             preferred_element_type=jnp.float32)


def _sc_body(tab_hbm, idx_hbm, out_hbm, ib, gb0, gb1, gb2, acc, sem0, sem1,
             sem2, sem3):
    wid = lax.axis_index("s") * 2 + lax.axis_index("c")
    gbs = (gb0, gb1, gb2)
    sems = (sem1, sem2, sem3)

    def _accum(buf):
        def _row(r):
            for cc in range(D // L):
                plsc.addupdate(acc.at[r, pl.ds(cc * L, L)],
                               buf[r, pl.ds(cc * L, L)])

        plsc.parallel_loop(0, SC_C, unroll=4)(_row)

    def chunk_body(ci, carry):
        chunk = wid + ci * NW

        @pl.when(chunk < NCHUNK)
        def _():
            pltpu.sync_copy(
                idx_hbm.at[pl.ds(chunk * K * SC_C, K * SC_C)], ib)

            def isl(k):
                return ib.at[pl.ds(k * SC_C, SC_C)]

            pltpu.async_copy(tab_hbm.at[isl(0)], acc, sem0)
            for k in range(1, 4):
                pltpu.async_copy(tab_hbm.at[isl(k)], gbs[k - 1],
                                 sems[k - 1])
            pltpu.make_async_copy(tab_hbm.at[isl(0)], acc, sem0).wait()
            for k in range(1, K):
                b = (k - 1) % 3
                pltpu.make_async_copy(tab_hbm.at[isl(k)], gbs[b],
                                      sems[b]).wait()
                _accum(gbs[b])
                if k + 3 < K:
                    pltpu.async_copy(tab_hbm.at[isl(k + 3)], gbs[b],
                                     sems[b])
            base = chunk * SC_C

            @pl.when(chunk < NCHUNK - 1)
            def _full():
                pltpu.sync_copy(acc, out_hbm.at[pl.ds(base, SC_C)])

            @pl.when(chunk == NCHUNK - 1)
            def _tail():
                pltpu.sync_copy(acc.at[pl.ds(0, TAIL)],
                                out_hbm.at[pl.ds(base, TAIL)])

        return carry

    lax.fori_loop(0, CPW, chunk_body, 0)


def kernel(lv_coarse, neighbor_idx, gn_gamma, gn_beta, weight):
    idx_pad = jnp.pad(neighbor_idx, ((0, NCHUNK * SC_C - N_FINE), (0, 0)))
    idx_cm = idx_pad.reshape(NCHUNK, SC_C, K).transpose(0, 2, 1).reshape(-1)
    idx_2d = jnp.pad(idx_cm, (0, IPAD - NIDX)).reshape(IROWS, 128)

    table, flat_idx = pl.pallas_call(
        _tc_body,
        grid=(NRB,),
        in_specs=[
            pl.BlockSpec((NCP, D), lambda i: (0, 0)),
            pl.BlockSpec((1, D), lambda i: (0, 0)),
            pl.BlockSpec((1, D), lambda i: (0, 0)),
            pl.BlockSpec((K * D, D), lambda i: (0, 0)),
            pl.BlockSpec((IROWS, 128), lambda i: (0, 0)),
        ],
        out_specs=[
            pl.BlockSpec((K, RB, D), lambda i: (0, i, 0)),
            pl.BlockSpec((IROWS, 128), lambda i: (0, 0)),
        ],
        out_shape=[
            jax.ShapeDtypeStruct((K, NCP, D), jnp.float32),
            jax.ShapeDtypeStruct((IROWS, 128), jnp.int32),
        ],
        scratch_shapes=[
            pltpu.VMEM((2, D), jnp.float32),
        ],
    )(lv_coarse, gn_gamma.reshape(1, D), gn_beta.reshape(1, D),
      weight.astype(jnp.bfloat16), idx_2d)

    tab_flat = table.reshape(K * NCP, D)
    flat_idx = flat_idx.reshape(IPAD)

    mesh = plsc.VectorSubcoreMesh(core_axis_name="c", subcore_axis_name="s")
    out = pl.kernel(
        _sc_body,
        out_type=jax.ShapeDtypeStruct((N_FINE, D), jnp.float32),
        mesh=mesh,
        scratch_types=[
            pltpu.VMEM((K * SC_C,), jnp.int32),
            pltpu.VMEM((SC_C, D), jnp.float32),
            pltpu.VMEM((SC_C, D), jnp.float32),
            pltpu.VMEM((SC_C, D), jnp.float32),
            pltpu.VMEM((SC_C, D), jnp.float32),
            pltpu.SemaphoreType.DMA,
            pltpu.SemaphoreType.DMA,
            pltpu.SemaphoreType.DMA,
            pltpu.SemaphoreType.DMA,
        ],
    )(tab_flat, flat_idx)
    return out


# speedup vs baseline: 2.3163x; 1.0007x over previous
import jax
import jax.numpy as jnp
from jax import lax
from jax.experimental import pallas as pl
from jax.experimental.pallas import tpu as pltpu
from jax.experimental.pallas import tpu_sc as plsc

N_COARSE = 12500
N_FINE = 25000
D = 256
K = 9
CG = 8
EPS = 1e-5

RB = 512
NCP = 12800
NRB = NCP // RB

NW = 32
L = 16
SC_C = 112
NCHUNK = -(-N_FINE // SC_C)
TAIL = N_FINE - (NCHUNK - 1) * SC_C
CPW = -(-NCHUNK // NW)

NIDX = NCHUNK * SC_C * K
IROWS = -(-NIDX // 128)
IPAD = IROWS * 128


def _tc_body(x_ref, g_ref, b_ref, w_ref, i2_ref, out_ref, o2_ref, sb_ref):
    i = pl.program_id(0)

    @pl.when(i == 0)
    def _stats():
        rows = lax.broadcasted_iota(jnp.int32, (NCP, 1), 0)
        x = jnp.where(rows < N_COARSE, x_ref[...], 0.0)
        s = jnp.sum(x, axis=0, keepdims=True)
        q = jnp.sum(x * x, axis=0, keepdims=True)
        row = lax.broadcasted_iota(jnp.int32, (D, D), 0) // CG
        col = lax.broadcasted_iota(jnp.int32, (D, D), 1) // CG
        m = (row == col).astype(jnp.float32)
        cnt = float(CG * N_COARSE)
        gs = jnp.dot(s, m, preferred_element_type=jnp.float32) / cnt
        gq = jnp.dot(q, m, preferred_element_type=jnp.float32) / cnt
        var = gq - gs * gs
        scale = g_ref[...] * lax.rsqrt(var + EPS)
        bias = b_ref[...] - gs * scale
        sb_ref[0:1, :] = scale
        sb_ref[1:2, :] = bias
        pos = (lax.broadcasted_iota(jnp.int32, (IROWS, 128), 0) * 128
               + lax.broadcasted_iota(jnp.int32, (IROWS, 128), 1))
        o2_ref[...] = i2_ref[...] + ((pos // SC_C) % K) * NCP

    scale = sb_ref[0:1, :]
    bias = sb_ref[1:2, :]
    x = x_ref[pl.ds(i * RB, RB), :]
    y = jnp.maximum(x * scale + bias, 0.0).astype(jnp.bfloat16)
    for k in range(K):
        out_ref[k] = jnp.dot(y, w_ref[pl.ds(k * D, D), :],
                             preferred_element_type=jnp.float32)


def _sc_body(tab_hbm, idx_hbm, out_hbm, ib, gb0, gb1, gb2, acc, sem0, sem1,
             sem2, sem3):
    wid = lax.axis_index("s") * 2 + lax.axis_index("c")
    gbs = (gb0, gb1, gb2)
    sems = (sem1, sem2, sem3)

    def _accum(buf):
        def _row(r):
            for cc in range(D // L):
                plsc.addupdate(acc.at[r, pl.ds(cc * L, L)],
                               buf[r, pl.ds(cc * L, L)])

        plsc.parallel_loop(0, SC_C, unroll=4)(_row)

    def chunk_body(ci, carry):
        chunk = wid + ci * NW

        @pl.when(chunk < NCHUNK)
        def _():
            pltpu.sync_copy(
                idx_hbm.at[pl.ds(chunk * K * SC_C, K * SC_C)], ib)

            def isl(k):
                return ib.at[pl.ds(k * SC_C, SC_C)]

            pltpu.async_copy(tab_hbm.at[isl(0)], acc, sem0)
            for k in range(1, 4):
                pltpu.async_copy(tab_hbm.at[isl(k)], gbs[k - 1],
                                 sems[k - 1])
            pltpu.make_async_copy(tab_hbm.at[isl(0)], acc, sem0).wait()
            for k in range(1, K):
                b = (k - 1) % 3
                pltpu.make_async_copy(tab_hbm.at[isl(k)], gbs[b],
                                      sems[b]).wait()
                _accum(gbs[b])
                if k + 3 < K:
                    pltpu.async_copy(tab_hbm.at[isl(k + 3)], gbs[b],
                                     sems[b])
            base = chunk * SC_C

            @pl.when(chunk < NCHUNK - 1)
            def _full():
                pltpu.sync_copy(acc, out_hbm.at[pl.ds(base, SC_C)])

            @pl.when(chunk == NCHUNK - 1)
            def _tail():
                pltpu.sync_copy(acc.at[pl.ds(0, TAIL)],
                                out_hbm.at[pl.ds(base, TAIL)])

        return carry

    lax.fori_loop(0, CPW, chunk_body, 0)


def kernel(lv_coarse, neighbor_idx, gn_gamma, gn_beta, weight):
    idx_pad = jnp.pad(neighbor_idx, ((0, NCHUNK * SC_C - N_FINE), (0, 0)))
    idx_cm = idx_pad.reshape(NCHUNK, SC_C, K).transpose(0, 2, 1).reshape(-1)
    idx_2d = jnp.pad(idx_cm, (0, IPAD - NIDX)).reshape(IROWS, 128)

    table, flat_idx = pl.pallas_call(
        _tc_body,
        grid=(NRB,),
        in_specs=[
            pl.BlockSpec((NCP, D), lambda i: (0, 0)),
            pl.BlockSpec((1, D), lambda i: (0, 0)),
            pl.BlockSpec((1, D), lambda i: (0, 0)),
            pl.BlockSpec((K * D, D), lambda i: (0, 0)),
            pl.BlockSpec((IROWS, 128), lambda i: (0, 0)),
        ],
        out_specs=[
            pl.BlockSpec((K, RB, D), lambda i: (0, i, 0)),
            pl.BlockSpec((IROWS, 128), lambda i: (0, 0)),
        ],
        out_shape=[
            jax.ShapeDtypeStruct((K, NCP, D), jnp.float32),
            jax.ShapeDtypeStruct((IROWS, 128), jnp.int32),
        ],
        scratch_shapes=[
            pltpu.VMEM((2, D), jnp.float32),
        ],
    )(lv_coarse, gn_gamma.reshape(1, D), gn_beta.reshape(1, D),
      weight.astype(jnp.bfloat16), idx_2d)

    tab_flat = table.reshape(K * NCP, D)
    flat_idx = flat_idx.reshape(IPAD)

    mesh = plsc.VectorSubcoreMesh(core_axis_name="c", subcore_axis_name="s")
    out = pl.kernel(
        _sc_body,
        out_type=jax.ShapeDtypeStruct((N_FINE, D), jnp.float32),
        mesh=mesh,
        scratch_types=[
            pltpu.VMEM((K * SC_C,), jnp.int32),
            pltpu.VMEM((SC_C, D), jnp.float32),
            pltpu.VMEM((SC_C, D), jnp.float32),
            pltpu.VMEM((SC_C, D), jnp.float32),
            pltpu.VMEM((SC_C, D), jnp.float32),
            pltpu.SemaphoreType.DMA,
            pltpu.SemaphoreType.DMA,
            pltpu.SemaphoreType.DMA,
            pltpu.SemaphoreType.DMA,
        ],
    )(tab_flat, flat_idx)
    return out
